# Initial kernel scaffold; baseline (speedup 1.0000x reference)
import jax
import jax.numpy as jnp
from jax.experimental import pallas as pl


def kernel(router_logits, correction_bias):
    raise NotImplementedError("write your pallas kernel here")


# speedup vs baseline: 3.5194x; 3.5194x over previous
import functools

import jax
import jax.numpy as jnp
from jax import lax
from jax.experimental import pallas as pl
from jax.experimental.pallas import tpu as pltpu
from jax.experimental.pallas import tpu_sc as plsc

NUM_TOKENS = 16384
NUM_EXPERTS = 64
NUM_GROUPS = 8
GROUP_SIZE = NUM_EXPERTS // NUM_GROUPS
TOPK_GROUPS = 4
K = 8
SCALE = 2.5

NC = 2
NS = 16
L = 16
NW = NC * NS
TPW = NUM_TOKENS // NW
TILE = L
NT = TPW // TILE


def _tec_kernel(logits_hbm, bias_hbm, w_hbm, id_hbm,
                raw_v, bias_v, wout_v, iout_v, sbuf_v):
    wid = lax.axis_index("s") * NC + lax.axis_index("c")
    base = wid * TPW

    pltpu.sync_copy(logits_hbm.at[pl.ds(base * NUM_EXPERTS, TPW * NUM_EXPERTS)],
                    raw_v)
    pltpu.sync_copy(bias_hbm, bias_v)

    iota = lax.iota(jnp.int32, L)
    neg_inf = jnp.full((L,), -jnp.inf, jnp.float32)
    bias_chunks = [bias_v[pl.ds(c * L, L)] for c in range(NUM_EXPERTS // L)]
    bias_s = [bias_chunks[e // L][e % L] for e in range(NUM_EXPERTS)]

    def tile_body(t, carry):
        rowbase = (t * TILE + iota) * NUM_EXPERTS

        m1 = [neg_inf] * NUM_GROUPS
        m2 = [neg_inf] * NUM_GROUPS
        for e in range(NUM_EXPERTS):
            s = plsc.load_gather(raw_v, [rowbase + e]) + bias_s[e]
            sbuf_v[pl.ds(e * L, L)] = s
            g = e // GROUP_SIZE
            m2[g] = jnp.maximum(m2[g], jnp.minimum(m1[g], s))
            m1[g] = jnp.maximum(m1[g], s)
        gs = [m1[g] + m2[g] for g in range(NUM_GROUPS)]

        sel = []
        for g in range(NUM_GROUPS):
            cnt = jnp.zeros((L,), jnp.int32)
            for h in range(NUM_GROUPS):
                if h == g:
                    continue
                beats = gs[h] > gs[g]
                if h < g:
                    beats = jnp.logical_or(beats, gs[h] == gs[g])
                cnt = cnt + jnp.where(beats, 1, 0)
            sel.append(cnt < TOPK_GROUPS)

        for e in range(NUM_EXPERTS):
            sbuf_v[pl.ds(e * L, L)] = jnp.where(
                sel[e // GROUP_SIZE], sbuf_v[pl.ds(e * L, L)], neg_inf)

        ws = []
        bis = []
        for k in range(K):
            bv = neg_inf
            bi = jnp.zeros((L,), jnp.int32)
            for e in range(NUM_EXPERTS):
                v = sbuf_v[pl.ds(e * L, L)]
                better = v > bv
                bv = jnp.where(better, v, bv)
                bi = jnp.where(better, jnp.full((L,), e, jnp.int32), bi)
            plsc.store_scatter(sbuf_v, [bi * L + iota], neg_inf)
            ws.append(plsc.load_gather(raw_v, [rowbase + bi]))
            bis.append(bi)

        wsum = ws[0]
        for k in range(1, K):
            wsum = wsum + ws[k]
        inv = SCALE / wsum
        outbase = (t * TILE + iota) * K
        for k in range(K):
            plsc.store_scatter(wout_v, [outbase + k], ws[k] * inv)
            plsc.store_scatter(iout_v, [outbase + k], bis[k])
        return carry

    lax.fori_loop(0, NT, tile_body, 0)

    pltpu.sync_copy(wout_v, w_hbm.at[pl.ds(base * K, TPW * K)])
    pltpu.sync_copy(iout_v, id_hbm.at[pl.ds(base * K, TPW * K)])


@jax.jit
def kernel(router_logits, correction_bias):
    mesh = plsc.VectorSubcoreMesh(core_axis_name="c", subcore_axis_name="s")
    run = functools.partial(
        pl.kernel,
        out_type=(
            jax.ShapeDtypeStruct((NUM_TOKENS * K,), jnp.float32),
            jax.ShapeDtypeStruct((NUM_TOKENS * K,), jnp.int32),
        ),
        mesh=mesh,
        compiler_params=pltpu.CompilerParams(needs_layout_passes=False),
        scratch_types=[
            pltpu.VMEM((TPW * NUM_EXPERTS,), jnp.float32),
            pltpu.VMEM((NUM_EXPERTS,), jnp.float32),
            pltpu.VMEM((TPW * K,), jnp.float32),
            pltpu.VMEM((TPW * K,), jnp.int32),
            pltpu.VMEM((NUM_EXPERTS * L,), jnp.float32),
        ],
    )(_tec_kernel)
    w_flat, id_flat = run(router_logits.reshape(-1), correction_bias)
    return (w_flat.reshape(NUM_TOKENS, K), id_flat.reshape(NUM_TOKENS, K))


# speedup vs baseline: 3.8007x; 1.0799x over previous
import functools

import jax
import jax.numpy as jnp
from jax import lax
from jax.experimental import pallas as pl
from jax.experimental.pallas import tpu as pltpu
from jax.experimental.pallas import tpu_sc as plsc

NUM_TOKENS = 16384
NUM_EXPERTS = 64
NUM_GROUPS = 8
GROUP_SIZE = NUM_EXPERTS // NUM_GROUPS
TOPK_GROUPS = 4
K = 8
SCALE = 2.5

NC = 2
NS = 16
L = 16
NW = NC * NS
TPW = NUM_TOKENS // NW
TILE = L
NT = TPW // TILE


def _tec_kernel(logits_hbm, bias_hbm, w_hbm, id_hbm,
                raw_v, bias_v, wout_v, iout_v, sbuf_v, cbuf_v, gmap_v):
    wid = lax.axis_index("s") * NC + lax.axis_index("c")
    base = wid * TPW

    pltpu.sync_copy(logits_hbm.at[pl.ds(base * NUM_EXPERTS, TPW * NUM_EXPERTS)],
                    raw_v)
    pltpu.sync_copy(bias_hbm, bias_v)

    iota = lax.iota(jnp.int32, L)
    neg_inf = jnp.full((L,), -jnp.inf, jnp.float32)
    bias_chunks = [bias_v[pl.ds(c * L, L)] for c in range(NUM_EXPERTS // L)]
    bias_s = [bias_chunks[e // L][e % L] for e in range(NUM_EXPERTS)]

    def tile_body(t, carry):
        rowbase = (t * TILE + iota) * NUM_EXPERTS

        m1 = [neg_inf] * NUM_GROUPS
        m2 = [neg_inf] * NUM_GROUPS
        for e in range(NUM_EXPERTS):
            s = plsc.load_gather(raw_v, [rowbase + e]) + bias_s[e]
            sbuf_v[pl.ds(e * L, L)] = s
            g = e // GROUP_SIZE
            m2[g] = jnp.maximum(m2[g], jnp.minimum(m1[g], s))
            m1[g] = jnp.maximum(m1[g], s)
        gs = [m1[g] + m2[g] for g in range(NUM_GROUPS)]

        sel = []
        for g in range(NUM_GROUPS):
            cnt = jnp.zeros((L,), jnp.int32)
            for h in range(NUM_GROUPS):
                if h == g:
                    continue
                beats = gs[h] > gs[g]
                if h < g:
                    beats = jnp.logical_or(beats, gs[h] == gs[g])
                cnt = cnt + jnp.where(beats, 1, 0)
            sel.append(cnt < TOPK_GROUPS)

        rank = jnp.zeros((L,), jnp.int32)
        gbase = []
        for g in range(NUM_GROUPS):
            gbase.append(rank * (GROUP_SIZE * L) + iota)
            plsc.store_scatter(gmap_v, [rank * L + iota],
                               jnp.full((L,), g, jnp.int32), mask=sel[g])
            rank = rank + jnp.where(sel[g], 1, 0)
        for e in range(NUM_EXPERTS):
            g = e // GROUP_SIZE
            plsc.store_scatter(cbuf_v, [gbase[g] + (e % GROUP_SIZE) * L],
                               sbuf_v[pl.ds(e * L, L)], mask=sel[g])

        NCAND = TOPK_GROUPS * GROUP_SIZE
        cand = [cbuf_v[pl.ds(i * L, L)] for i in range(NCAND)]
        ws = []
        bis = []
        for k in range(K):
            vals = list(cand)
            idxs = [jnp.full((L,), i, jnp.int32) for i in range(NCAND)]
            n = NCAND
            while n > 1:
                nv, ni = [], []
                for i in range(0, n, 2):
                    better = vals[i + 1] > vals[i]
                    nv.append(jnp.where(better, vals[i + 1], vals[i]))
                    ni.append(jnp.where(better, idxs[i + 1], idxs[i]))
                vals, idxs, n = nv, ni, n // 2
            bslot = idxs[0]
            for i in range(NCAND):
                cand[i] = jnp.where(bslot == i, neg_inf, cand[i])
            gm = plsc.load_gather(gmap_v, [(bslot // GROUP_SIZE) * L + iota])
            bi = gm * GROUP_SIZE + (bslot % GROUP_SIZE)
            ws.append(plsc.load_gather(raw_v, [rowbase + bi]))
            bis.append(bi)

        wsum = ws[0]
        for k in range(1, K):
            wsum = wsum + ws[k]
        inv = SCALE / wsum
        outbase = (t * TILE + iota) * K
        for k in range(K):
            plsc.store_scatter(wout_v, [outbase + k], ws[k] * inv)
            plsc.store_scatter(iout_v, [outbase + k], bis[k])
        return carry

    lax.fori_loop(0, NT, tile_body, 0)

    pltpu.sync_copy(wout_v, w_hbm.at[pl.ds(base * K, TPW * K)])
    pltpu.sync_copy(iout_v, id_hbm.at[pl.ds(base * K, TPW * K)])


@jax.jit
def kernel(router_logits, correction_bias):
    mesh = plsc.VectorSubcoreMesh(core_axis_name="c", subcore_axis_name="s")
    run = functools.partial(
        pl.kernel,
        out_type=(
            jax.ShapeDtypeStruct((NUM_TOKENS * K,), jnp.float32),
            jax.ShapeDtypeStruct((NUM_TOKENS * K,), jnp.int32),
        ),
        mesh=mesh,
        compiler_params=pltpu.CompilerParams(needs_layout_passes=False),
        scratch_types=[
            pltpu.VMEM((TPW * NUM_EXPERTS,), jnp.float32),
            pltpu.VMEM((NUM_EXPERTS,), jnp.float32),
            pltpu.VMEM((TPW * K,), jnp.float32),
            pltpu.VMEM((TPW * K,), jnp.int32),
            pltpu.VMEM((NUM_EXPERTS * L,), jnp.float32),
            pltpu.VMEM((TOPK_GROUPS * GROUP_SIZE * L,), jnp.float32),
            pltpu.VMEM((TOPK_GROUPS * L,), jnp.int32),
        ],
    )(_tec_kernel)
    w_flat, id_flat = run(router_logits.reshape(-1), correction_bias)
    return (w_flat.reshape(NUM_TOKENS, K), id_flat.reshape(NUM_TOKENS, K))
